# baseline (speedup 1.0000x reference)
import functools

import jax
import jax.numpy as jnp
from jax import lax
from jax.experimental import pallas as pl
from jax.experimental.pallas import tpu as pltpu
from jax.experimental.pallas import tpu_sc as plsc

_L = 16
_NC, _NS = 2, 16
_NW = _NC * _NS


def _mlp_table_kernel(G, x_ref, f_ref, w1x_ref, w1f_ref, b1_ref, w2_ref,
                      b2_ref, w3_ref, b3_ref, out_ref):
  pid = pl.program_id(0)

  @pl.when(pid < G)
  def _():
    bf = jnp.bfloat16
    h = jnp.dot(x_ref[...].astype(bf), w1x_ref[...],
                preferred_element_type=jnp.float32)
    h += jnp.dot(f_ref[...].astype(bf), w1f_ref[...],
                 preferred_element_type=jnp.float32)
    h = jnp.maximum(h + b1_ref[...], 0.0).astype(bf)
    h = jnp.maximum(
        jnp.dot(h, w2_ref[...], preferred_element_type=jnp.float32)
        + b2_ref[...], 0.0).astype(bf)
    out_ref[...] = (
        jnp.dot(h, w3_ref[...], preferred_element_type=jnp.float32)
        + b3_ref[...])

  @pl.when(pid == G)
  def _():
    out_ref[...] = jnp.zeros_like(out_ref)

  @pl.when(pid > G)
  def _():
    out_ref[...] = f_ref[...]


def _shift_up_fn(iota):
  gd = lax.GatherDimensionNumbers(
      offset_dims=(), collapsed_slice_dims=(0,), start_index_map=(0,))

  def shift_up(v):
    idx = jnp.minimum(iota + 1, _L - 1)
    return lax.gather(v, idx[:, None], gd, slice_sizes=(1,),
                      mode=lax.GatherScatterMode.PROMISE_IN_BOUNDS)
  return shift_up


def _make_scan_kernel(NT, NP, NSEC=4):
  mesh = plsc.VectorSubcoreMesh(core_axis_name="c", subcore_axis_name="s",
                                num_cores=_NC, num_subcores=_NS)
  TPW = NT // _NW
  SEC = NP // NSEC
  _MAXI = 0x7FFFFFFF

  def body(fa_hbm, lc_hbm, rc_hbm, wt_hbm, wl_hbm, wr_hbm,
           wt_ref, wl_ref, wr_ref, fch, lch, rch, sem):
    cid = lax.axis_index("c")
    sid = lax.axis_index("s")
    wid = sid * _NC + cid
    iota = lax.iota(jnp.int32, _L)
    shift_up = _shift_up_fn(iota)

    with jax.named_scope("ph_stage"):
      pltpu.sync_copy(fa_hbm.at[pl.ds(wid * TPW, TPW)], fch)
      pltpu.sync_copy(lc_hbm.at[pl.ds(wid * TPW, TPW)], lch)
      pltpu.sync_copy(rc_hbm.at[pl.ds(wid * TPW, TPW)], rch)
    t_off = wid * TPW

    for sec in range(NSEC):
      sec_lo = sec * SEC
      with jax.named_scope("ph_memset"):
        @pl.loop(0, SEC // _L)
        def _(i):
          wt_ref[pl.ds(i * _L, _L)] = jnp.full((_L,), -1, jnp.int32)

      with jax.named_scope("ph_localscan"):
        @pl.loop(0, TPW // _L)
        def _(ic):
          s = pl.ds(ic * _L, _L)
          fl = fch[s] - sec_lo
          valid = (fl >= 0) & (fl < SEC)
          key = jnp.where(valid, lax.shift_left(fl, 12) | (ic * _L + iota),
                          _MAXI)
          ks, lv = plsc.sort_key_val(key, lch[s])
          _, rv = plsc.sort_key_val(key, rch[s])
          fs = lax.shift_right_logical(ks, 12)
          is_last = (fs != shift_up(fs)) | (iota == _L - 1)
          m = is_last & (ks != _MAXI)
          plsc.store_scatter(wt_ref, [fs], (ks & 0xFFF) + t_off, mask=m)
          plsc.store_scatter(wl_ref, [fs], lv, mask=m)
          plsc.store_scatter(wr_ref, [fs], rv, mask=m)

      with jax.named_scope("ph_publish"):
        pltpu.sync_copy(wt_ref, wt_hbm.at[wid, pl.ds(sec_lo, SEC)])
        pltpu.sync_copy(wl_ref, wl_hbm.at[wid, pl.ds(sec_lo, SEC)])
        pltpu.sync_copy(wr_ref, wr_hbm.at[wid, pl.ds(sec_lo, SEC)])

  sd = jax.ShapeDtypeStruct((_NW, NP), jnp.int32)
  return pl.kernel(
      body,
      out_type=(sd, sd, sd),
      mesh=mesh,
      compiler_params=pltpu.CompilerParams(needs_layout_passes=False),
      scratch_types=[
          pltpu.VMEM((SEC,), jnp.int32),
          pltpu.VMEM((SEC,), jnp.int32),
          pltpu.VMEM((SEC,), jnp.int32),
          pltpu.VMEM((TPW,), jnp.int32),
          pltpu.VMEM((TPW,), jnp.int32),
          pltpu.VMEM((TPW,), jnp.int32),
          pltpu.SemaphoreType.DMA,
      ],
  )


def _make_merge_kernel(NP, T, ZBASE, FBASE, RPT):
  mesh = plsc.VectorSubcoreMesh(core_axis_name="c", subcore_axis_name="s",
                                num_cores=_NC, num_subcores=_NS)
  NQ = 5
  Q = RPT // NQ

  def body(wt_hbm, wl_hbm, wr_hbm, a_hbm, b_hbm,
           a_ref, b_ref, mt, ml, mr, semm):
    cid = lax.axis_index("c")
    sid = lax.axis_index("s")
    wid = sid * _NC + cid
    lo = wid * RPT
    iota = lax.iota(jnp.int32, _L)

    with jax.named_scope("ph_merge"):
      for q in range(NQ):
        qo = q * Q
        src_t = wt_hbm.at[pl.ds(0, _NW), pl.ds(lo + qo, Q)]
        src_l = wl_hbm.at[pl.ds(0, _NW), pl.ds(lo + qo, Q)]
        src_r = wr_hbm.at[pl.ds(0, _NW), pl.ds(lo + qo, Q)]
        pltpu.async_copy(src_t, mt, semm)
        pltpu.async_copy(src_l, ml, semm)
        pltpu.async_copy(src_r, mr, semm)
        pltpu.make_async_copy(src_t, mt, semm).wait()
        pltpu.make_async_copy(src_l, ml, semm).wait()
        pltpu.make_async_copy(src_r, mr, semm).wait()

        @pl.loop(0, Q // _L)
        def _(i):
          s = pl.ds(i * _L, _L)
          at = mt[0, s]
          al = ml[0, s]
          ar = mr[0, s]
          for k in range(1, _NW):
            tk = mt[k, s]
            m = tk > at
            at = jnp.where(m, tk, at)
            al = jnp.where(m, ml[k, s], al)
            ar = jnp.where(m, mr[k, s], ar)
          j = lo + qo + i * _L + iota
          a_ref[pl.ds(qo + i * _L, _L)] = jnp.where(at < 0, FBASE + j, al)
          b_ref[pl.ds(qo + i * _L, _L)] = jnp.where(
              at < 0, ZBASE + (j & (T - 1)), ar)

      pltpu.sync_copy(a_ref, a_hbm.at[pl.ds(lo, RPT)])
      pltpu.sync_copy(b_ref, b_hbm.at[pl.ds(lo, RPT)])

  sd = jax.ShapeDtypeStruct((NP,), jnp.int32)
  return pl.kernel(
      body,
      out_type=(sd, sd),
      mesh=mesh,
      compiler_params=pltpu.CompilerParams(needs_layout_passes=False),
      scratch_types=[
          pltpu.VMEM((RPT,), jnp.int32),
          pltpu.VMEM((RPT,), jnp.int32),
          pltpu.VMEM((_NW, RPT // 5), jnp.int32),
          pltpu.VMEM((_NW, RPT // 5), jnp.int32),
          pltpu.VMEM((_NW, RPT // 5), jnp.int32),
          pltpu.SemaphoreType.DMA,
      ],
  )


def _make_gather_kernel(N, D, NP, RPT, C):
  mesh = plsc.VectorSubcoreMesh(core_axis_name="c", subcore_axis_name="s",
                                num_cores=_NC, num_subcores=_NS)
  P = 4

  def body(a_hbm, b_hbm, mf_hbm, out_hbm,
           a_ref, b_ref,
           abuf0, abuf1, abuf2, abuf3, sem0, sem1, sem2, sem3,
           semw0, semw1, semw2, semw3):
    abuf = (abuf0, abuf1, abuf2, abuf3)
    sem = (sem0, sem1, sem2, sem3)
    semw = (semw0, semw1, semw2, semw3)
    cid = lax.axis_index("c")
    sid = lax.axis_index("s")
    wid = sid * _NC + cid
    lo = wid * RPT

    with jax.named_scope("ph_ldidx"):
      pltpu.sync_copy(a_hbm.at[pl.ds(lo, RPT)], a_ref)
      pltpu.sync_copy(b_hbm.at[pl.ds(lo, RPT)], b_ref)

    NCH = RPT // C

    def fire_a(c, par):
      pltpu.async_copy(
          mf_hbm.at[a_ref.at[pl.ds(c * C, C)]], abuf[par], sem[par])

    def fire_b(c, par):
      pltpu.async_copy(
          mf_hbm.at[b_ref.at[pl.ds(c * C, C)]], abuf[par], sem[par],
          add=True)

    def drain(par):
      pltpu.make_async_copy(
          mf_hbm.at[a_ref.at[pl.ds(0, C)]], abuf[par], sem[par]).wait()

    def drain_w(par):
      pltpu.make_async_copy(
          abuf[par], out_hbm.at[pl.ds(0, C)], semw[par]).wait()

    with jax.named_scope("ph_gather"):
      for par in range(P):
        fire_a(par, par)

      @pl.loop(0, NCH // P)
      def _(oc):
        for par in range(P):
          c = oc * P + par
          drain(par)
          fire_b(c, par)
          drain(par)

          @pl.when(lo + c * C + C <= N)
          def _():
            pltpu.async_copy(
                abuf[par], out_hbm.at[pl.ds(lo + c * C, C)], semw[par])

          @pl.when(c + P < NCH)
          def _():
            @pl.when(lo + c * C + C <= N)
            def _():
              drain_w(par)
            fire_a(c + P, par)

  return pl.kernel(
      body,
      out_type=jax.ShapeDtypeStruct((N, D), jnp.float32),
      mesh=mesh,
      compiler_params=pltpu.CompilerParams(needs_layout_passes=False),
      scratch_types=[
          pltpu.VMEM((RPT,), jnp.int32),
          pltpu.VMEM((RPT,), jnp.int32),
          pltpu.VMEM((C, D), jnp.float32),
          pltpu.VMEM((C, D), jnp.float32),
          pltpu.VMEM((C, D), jnp.float32),
          pltpu.VMEM((C, D), jnp.float32),
          pltpu.SemaphoreType.DMA,
          pltpu.SemaphoreType.DMA,
          pltpu.SemaphoreType.DMA,
          pltpu.SemaphoreType.DMA,
          pltpu.SemaphoreType.DMA,
          pltpu.SemaphoreType.DMA,
          pltpu.SemaphoreType.DMA,
          pltpu.SemaphoreType.DMA,
      ],
  )


@jax.jit
def kernel(X, Feature, I_list, W1, b1, W2, b2, W3, b3):
  N, D = Feature.shape
  nlev, _, ni, _ = I_list.shape
  NT = nlev * ni
  T = 4096
  NP = -(-N // 4096) * 4096
  G = NP // T
  ZBASE = NP
  FBASE = NP + T
  RPT = NP // _NW
  C = 80
  NB = -(-N // T)

  w1x = W1[:6].astype(jnp.bfloat16)
  w1f = W1[6:].astype(jnp.bfloat16)
  w2b = W2.astype(jnp.bfloat16)
  w3b = W3.astype(jnp.bfloat16)
  b1r = b1.reshape(1, D)
  b2r = b2.reshape(1, D)
  b3r = b3.reshape(1, D)
  mf = pl.pallas_call(
      functools.partial(_mlp_table_kernel, G),
      grid=(2 * G + 1,),
      in_specs=[
          pl.BlockSpec((T, 6), lambda i: (jnp.minimum(i, NB - 1), 0)),
          pl.BlockSpec((T, D), lambda i: (
              jnp.minimum(jnp.where(i < G, i, jnp.maximum(i - G - 1, 0)),
                          NB - 1), 0)),
          pl.BlockSpec((6, D), lambda i: (0, 0)),
          pl.BlockSpec((D, D), lambda i: (0, 0)),
          pl.BlockSpec((1, D), lambda i: (0, 0)),
          pl.BlockSpec((D, D), lambda i: (0, 0)),
          pl.BlockSpec((1, D), lambda i: (0, 0)),
          pl.BlockSpec((D, D), lambda i: (0, 0)),
          pl.BlockSpec((1, D), lambda i: (0, 0)),
      ],
      out_specs=pl.BlockSpec((T, D), lambda i: (i, 0)),
      out_shape=jax.ShapeDtypeStruct(((2 * G + 1) * T, D), jnp.float32),
  )(X, Feature, w1x, w1f, b1r, w2b, b2r, w3b, b3r)

  tri = I_list.reshape(NT, 3)
  father = tri[:, 2]
  lchild = tri[:, 0]
  rchild = tri[:, 1]
  wt, wl, wr = _make_scan_kernel(NT, NP)(father, lchild, rchild)
  a_idx, b_idx = _make_merge_kernel(NP, T, ZBASE, FBASE, RPT)(wt, wl, wr)

  return _make_gather_kernel(N, D, NP, RPT, C)(a_idx, b_idx, mf)


# speedup vs baseline: 1.1215x; 1.1215x over previous
import functools

import jax
import jax.numpy as jnp
from jax import lax
from jax.experimental import pallas as pl
from jax.experimental.pallas import tpu as pltpu
from jax.experimental.pallas import tpu_sc as plsc

_L = 16
_NC, _NS = 2, 16
_NW = _NC * _NS


def _mlp_table_kernel(G, x_ref, f_ref, w1x_ref, w1f_ref, b1_ref, w2_ref,
                      b2_ref, w3_ref, b3_ref, out_ref):
  pid = pl.program_id(0)

  @pl.when(pid < G)
  def _():
    bf = jnp.bfloat16
    h = jnp.dot(x_ref[...].astype(bf), w1x_ref[...],
                preferred_element_type=jnp.float32)
    h += jnp.dot(f_ref[...].astype(bf), w1f_ref[...],
                 preferred_element_type=jnp.float32)
    h = jnp.maximum(h + b1_ref[...], 0.0).astype(bf)
    h = jnp.maximum(
        jnp.dot(h, w2_ref[...], preferred_element_type=jnp.float32)
        + b2_ref[...], 0.0).astype(bf)
    out_ref[...] = (
        jnp.dot(h, w3_ref[...], preferred_element_type=jnp.float32)
        + b3_ref[...])

  @pl.when(pid == G)
  def _():
    out_ref[...] = jnp.zeros_like(out_ref)

  @pl.when(pid > G)
  def _():
    out_ref[...] = f_ref[...]


def _shift_up_fn(iota):
  gd = lax.GatherDimensionNumbers(
      offset_dims=(), collapsed_slice_dims=(0,), start_index_map=(0,))

  def shift_up(v):
    idx = jnp.minimum(iota + 1, _L - 1)
    return lax.gather(v, idx[:, None], gd, slice_sizes=(1,),
                      mode=lax.GatherScatterMode.PROMISE_IN_BOUNDS)
  return shift_up


def _make_scan_kernel(NT, NP, NSEC=4):
  mesh = plsc.VectorSubcoreMesh(core_axis_name="c", subcore_axis_name="s",
                                num_cores=_NC, num_subcores=_NS)
  TPW = NT // _NW
  SEC = NP // NSEC
  _MAXI = 0x7FFFFFFF

  def body(fa_hbm, lc_hbm, rc_hbm, wt_hbm, wl_hbm, wr_hbm,
           wt_ref, wl_ref, wr_ref, fch, lch, rch, sem):
    cid = lax.axis_index("c")
    sid = lax.axis_index("s")
    wid = sid * _NC + cid
    iota = lax.iota(jnp.int32, _L)
    shift_up = _shift_up_fn(iota)

    with jax.named_scope("ph_stage"):
      pltpu.sync_copy(fa_hbm.at[pl.ds(wid * TPW, TPW)], fch)
      pltpu.sync_copy(lc_hbm.at[pl.ds(wid * TPW, TPW)], lch)
      pltpu.sync_copy(rc_hbm.at[pl.ds(wid * TPW, TPW)], rch)
    t_off = wid * TPW

    for sec in range(NSEC):
      sec_lo = sec * SEC
      with jax.named_scope("ph_memset"):
        @pl.loop(0, SEC // _L)
        def _(i):
          wt_ref[pl.ds(i * _L, _L)] = jnp.full((_L,), -1, jnp.int32)

      with jax.named_scope("ph_localscan"):
        @pl.loop(0, TPW // _L)
        def _(ic):
          s = pl.ds(ic * _L, _L)
          fl = fch[s] - sec_lo
          valid = (fl >= 0) & (fl < SEC)
          key = jnp.where(valid, lax.shift_left(fl, 12) | (ic * _L + iota),
                          _MAXI)
          ks, lv = plsc.sort_key_val(key, lch[s])
          _, rv = plsc.sort_key_val(key, rch[s])
          fs = lax.shift_right_logical(ks, 12)
          is_last = (fs != shift_up(fs)) | (iota == _L - 1)
          m = is_last & (ks != _MAXI)
          plsc.store_scatter(wt_ref, [fs], (ks & 0xFFF) + t_off, mask=m)
          plsc.store_scatter(wl_ref, [fs], lv, mask=m)
          plsc.store_scatter(wr_ref, [fs], rv, mask=m)

      with jax.named_scope("ph_publish"):
        pltpu.sync_copy(wt_ref, wt_hbm.at[wid, pl.ds(sec_lo, SEC)])
        pltpu.sync_copy(wl_ref, wl_hbm.at[wid, pl.ds(sec_lo, SEC)])
        pltpu.sync_copy(wr_ref, wr_hbm.at[wid, pl.ds(sec_lo, SEC)])

  sd = jax.ShapeDtypeStruct((_NW, NP), jnp.int32)
  return pl.kernel(
      body,
      out_type=(sd, sd, sd),
      mesh=mesh,
      compiler_params=pltpu.CompilerParams(needs_layout_passes=False),
      scratch_types=[
          pltpu.VMEM((SEC,), jnp.int32),
          pltpu.VMEM((SEC,), jnp.int32),
          pltpu.VMEM((SEC,), jnp.int32),
          pltpu.VMEM((TPW,), jnp.int32),
          pltpu.VMEM((TPW,), jnp.int32),
          pltpu.VMEM((TPW,), jnp.int32),
          pltpu.SemaphoreType.DMA,
      ],
  )


def _make_gather_kernel(N, D, NP, T, ZBASE, FBASE, RPT, C):
  mesh = plsc.VectorSubcoreMesh(core_axis_name="c", subcore_axis_name="s",
                                num_cores=_NC, num_subcores=_NS)
  NQ = 5
  Q = RPT // NQ
  P = 4

  def body(wt_hbm, wl_hbm, wr_hbm, mf_hbm, out_hbm,
           a_ref, b_ref, mt, ml, mr, semm,
           abuf0, abuf1, abuf2, abuf3, sem0, sem1, sem2, sem3,
           semw0, semw1, semw2, semw3):
    abuf = (abuf0, abuf1, abuf2, abuf3)
    sem = (sem0, sem1, sem2, sem3)
    semw = (semw0, semw1, semw2, semw3)
    cid = lax.axis_index("c")
    sid = lax.axis_index("s")
    wid = sid * _NC + cid
    lo = wid * RPT
    iota = lax.iota(jnp.int32, _L)

    with jax.named_scope("ph_merge"):
      for q in range(NQ):
        qo = q * Q
        src_t = wt_hbm.at[pl.ds(0, _NW), pl.ds(lo + qo, Q)]
        src_l = wl_hbm.at[pl.ds(0, _NW), pl.ds(lo + qo, Q)]
        src_r = wr_hbm.at[pl.ds(0, _NW), pl.ds(lo + qo, Q)]
        pltpu.async_copy(src_t, mt, semm)
        pltpu.async_copy(src_l, ml, semm)
        pltpu.async_copy(src_r, mr, semm)
        pltpu.make_async_copy(src_t, mt, semm).wait()
        pltpu.make_async_copy(src_l, ml, semm).wait()
        pltpu.make_async_copy(src_r, mr, semm).wait()

        @pl.loop(0, Q // _L)
        def _(i):
          s = pl.ds(i * _L, _L)
          at = mt[0, s]
          al = ml[0, s]
          ar = mr[0, s]
          for k in range(1, _NW):
            tk = mt[k, s]
            m = tk > at
            at = jnp.where(m, tk, at)
            al = jnp.where(m, ml[k, s], al)
            ar = jnp.where(m, mr[k, s], ar)
          j = lo + qo + i * _L + iota
          a_ref[pl.ds(qo + i * _L, _L)] = jnp.where(at < 0, FBASE + j, al)
          b_ref[pl.ds(qo + i * _L, _L)] = jnp.where(
              at < 0, ZBASE + (j & (T - 1)), ar)

    NCH = RPT // C

    def fire_a(c, par):
      pltpu.async_copy(
          mf_hbm.at[a_ref.at[pl.ds(c * C, C)]], abuf[par], sem[par])

    def fire_b(c, par):
      pltpu.async_copy(
          mf_hbm.at[b_ref.at[pl.ds(c * C, C)]], abuf[par], sem[par],
          add=True)

    def drain(par):
      pltpu.make_async_copy(
          mf_hbm.at[a_ref.at[pl.ds(0, C)]], abuf[par], sem[par]).wait()

    def drain_w(par):
      pltpu.make_async_copy(
          abuf[par], out_hbm.at[pl.ds(0, C)], semw[par]).wait()

    with jax.named_scope("ph_gather"):
      for par in range(P):
        fire_a(par, par)

      @pl.loop(0, NCH // P)
      def _(oc):
        for par in range(P):
          c = oc * P + par
          drain(par)
          fire_b(c, par)
          drain(par)

          @pl.when(lo + c * C + C <= N)
          def _():
            pltpu.async_copy(
                abuf[par], out_hbm.at[pl.ds(lo + c * C, C)], semw[par])

          @pl.when(c + P < NCH)
          def _():
            @pl.when(lo + c * C + C <= N)
            def _():
              drain_w(par)
            fire_a(c + P, par)

  return pl.kernel(
      body,
      out_type=jax.ShapeDtypeStruct((N, D), jnp.float32),
      mesh=mesh,
      compiler_params=pltpu.CompilerParams(needs_layout_passes=False),
      scratch_types=[
          pltpu.VMEM((RPT,), jnp.int32),
          pltpu.VMEM((RPT,), jnp.int32),
          pltpu.VMEM((_NW, RPT // 5), jnp.int32),
          pltpu.VMEM((_NW, RPT // 5), jnp.int32),
          pltpu.VMEM((_NW, RPT // 5), jnp.int32),
          pltpu.SemaphoreType.DMA,
          pltpu.VMEM((C, D), jnp.float32),
          pltpu.VMEM((C, D), jnp.float32),
          pltpu.VMEM((C, D), jnp.float32),
          pltpu.VMEM((C, D), jnp.float32),
          pltpu.SemaphoreType.DMA,
          pltpu.SemaphoreType.DMA,
          pltpu.SemaphoreType.DMA,
          pltpu.SemaphoreType.DMA,
          pltpu.SemaphoreType.DMA,
          pltpu.SemaphoreType.DMA,
          pltpu.SemaphoreType.DMA,
          pltpu.SemaphoreType.DMA,
      ],
  )


@jax.jit
def kernel(X, Feature, I_list, W1, b1, W2, b2, W3, b3):
  N, D = Feature.shape
  nlev, _, ni, _ = I_list.shape
  NT = nlev * ni
  T = 4096
  NP = -(-N // 4096) * 4096
  G = NP // T
  ZBASE = NP
  FBASE = NP + T
  RPT = NP // _NW
  C = 80
  NB = -(-N // T)

  w1x = W1[:6].astype(jnp.bfloat16)
  w1f = W1[6:].astype(jnp.bfloat16)
  w2b = W2.astype(jnp.bfloat16)
  w3b = W3.astype(jnp.bfloat16)
  b1r = b1.reshape(1, D)
  b2r = b2.reshape(1, D)
  b3r = b3.reshape(1, D)
  mf = pl.pallas_call(
      functools.partial(_mlp_table_kernel, G),
      grid=(2 * G + 1,),
      in_specs=[
          pl.BlockSpec((T, 6), lambda i: (jnp.minimum(i, NB - 1), 0)),
          pl.BlockSpec((T, D), lambda i: (
              jnp.minimum(jnp.where(i < G, i, jnp.maximum(i - G - 1, 0)),
                          NB - 1), 0)),
          pl.BlockSpec((6, D), lambda i: (0, 0)),
          pl.BlockSpec((D, D), lambda i: (0, 0)),
          pl.BlockSpec((1, D), lambda i: (0, 0)),
          pl.BlockSpec((D, D), lambda i: (0, 0)),
          pl.BlockSpec((1, D), lambda i: (0, 0)),
          pl.BlockSpec((D, D), lambda i: (0, 0)),
          pl.BlockSpec((1, D), lambda i: (0, 0)),
      ],
      out_specs=pl.BlockSpec((T, D), lambda i: (i, 0)),
      out_shape=jax.ShapeDtypeStruct(((2 * G + 1) * T, D), jnp.float32),
  )(X, Feature, w1x, w1f, b1r, w2b, b2r, w3b, b3r)

  tri = I_list.reshape(NT, 3)
  father = tri[:, 2]
  lchild = tri[:, 0]
  rchild = tri[:, 1]
  wt, wl, wr = _make_scan_kernel(NT, NP)(father, lchild, rchild)

  return _make_gather_kernel(N, D, NP, T, ZBASE, FBASE, RPT, C)(
      wt, wl, wr, mf)
